# baseline (speedup 1.0000x reference)
import functools

import jax
import jax.numpy as jnp
from jax import lax
from jax.experimental import pallas as pl
from jax.experimental.pallas import tpu as pltpu
from jax.experimental.pallas import tpu_sc as plsc

VOCAB = 1000
VPAD = 1024
NC = 2
NS = 16
NW = NC * NS


def _sc_gather(idx4, table_p, b, t, tp):
    nb = b // NW
    mesh = plsc.VectorSubcoreMesh(core_axis_name="c", subcore_axis_name="s")

    @functools.partial(
        pl.kernel,
        mesh=mesh,
        out_type=jax.ShapeDtypeStruct((b, t, VOCAB), jnp.float32),
        scratch_types=[
            pltpu.VMEM((1, tp), jnp.int32),
            pltpu.VMEM((tp, VPAD), jnp.float32),
            pltpu.VMEM((t, VOCAB), jnp.float32),
            pltpu.SemaphoreType.DMA,
            pltpu.SemaphoreType.DMA,
            pltpu.SemaphoreType.DMA,
        ],
    )
    def k(idx_hbm, table_hbm, out_hbm, idx_v, gbuf, abuf, gsem, osem, isem):
        wid = lax.axis_index("s") * NC + lax.axis_index("c")
        base = wid * nb

        def gather_wait():
            pltpu.make_async_copy(table_hbm.at[idx_v.at[0]], gbuf, gsem).wait()

        def out_wait(bb):
            pltpu.make_async_copy(abuf, out_hbm.at[bb], osem).wait()

        pltpu.sync_copy(idx_hbm.at[base], idx_v)
        pltpu.async_copy(table_hbm.at[idx_v.at[0]], gbuf, gsem)

        def body(g, _):
            bb = base + g
            gather_wait()

            @pl.when(g < nb - 1)
            def _():
                pltpu.async_copy(idx_hbm.at[bb + 1], idx_v, isem)

            @pl.when(g > 0)
            def _():
                out_wait(bb - 1)

            def stitch(r, _):
                for off in [16 * j for j in range(62)] + [VOCAB - 16]:
                    abuf[r, pl.ds(off, 16)] = gbuf[r, pl.ds(off, 16)]
                return 0

            lax.fori_loop(0, t, stitch, 0)
            pltpu.async_copy(abuf, out_hbm.at[bb], osem)

            @pl.when(g < nb - 1)
            def _():
                pltpu.make_async_copy(idx_hbm.at[bb + 1], idx_v, isem).wait()
                pltpu.async_copy(table_hbm.at[idx_v.at[0]], gbuf, gsem)

            return 0

        lax.fori_loop(0, nb, body, 0)
        out_wait(base + nb - 1)

    return k(idx4, table_p)


def kernel(idx, table):
    b, t = idx.shape
    bs = b // 2
    tp = (t + 7) // 8 * 8
    idx4 = jnp.pad(idx[:bs].reshape(bs, 1, t), ((0, 0), (0, 0), (0, tp - t)))
    table_p = jnp.pad(table, ((0, 0), (0, VPAD - VOCAB)))
    out_a = _sc_gather(idx4, table_p, bs, t, tp)
    out_b = jnp.take(table, idx[bs:], axis=0)
    return jnp.concatenate([out_a, out_b], axis=0)


# speedup vs baseline: 1.2354x; 1.2354x over previous
import functools

import jax
import jax.numpy as jnp
from jax import lax
from jax.experimental import pallas as pl
from jax.experimental.pallas import tpu as pltpu
from jax.experimental.pallas import tpu_sc as plsc

VOCAB = 1000
VPAD = 1024
NC = 2
NS = 16
NW = NC * NS


def _sc_gather(idx4, table_p, b, t, tp):
    nb = b // NW
    mesh = plsc.VectorSubcoreMesh(core_axis_name="c", subcore_axis_name="s")

    @functools.partial(
        pl.kernel,
        mesh=mesh,
        out_type=jax.ShapeDtypeStruct((b, t, VOCAB), jnp.float32),
        scratch_types=[
            pltpu.VMEM((1, tp), jnp.int32),
            pltpu.VMEM((tp, VPAD), jnp.float32),
            pltpu.VMEM((t, VOCAB), jnp.float32),
            pltpu.SemaphoreType.DMA,
            pltpu.SemaphoreType.DMA,
            pltpu.SemaphoreType.DMA,
        ],
    )
    def k(idx_hbm, table_hbm, out_hbm, idx_v, gbuf, abuf, gsem, osem, isem):
        wid = lax.axis_index("s") * NC + lax.axis_index("c")
        base = wid * nb

        def gather_wait():
            pltpu.make_async_copy(table_hbm.at[idx_v.at[0]], gbuf, gsem).wait()

        def out_wait(bb):
            pltpu.make_async_copy(abuf, out_hbm.at[bb], osem).wait()

        pltpu.sync_copy(idx_hbm.at[base], idx_v)
        pltpu.async_copy(table_hbm.at[idx_v.at[0]], gbuf, gsem)

        def body(g, _):
            bb = base + g
            gather_wait()

            @pl.when(g < nb - 1)
            def _():
                pltpu.async_copy(idx_hbm.at[bb + 1], idx_v, isem)

            @pl.when(g > 0)
            def _():
                out_wait(bb - 1)

            def stitch(r, _):
                for off in [16 * j for j in range(62)] + [VOCAB - 16]:
                    abuf[r, pl.ds(off, 16)] = gbuf[r, pl.ds(off, 16)]
                return 0

            lax.fori_loop(0, t, stitch, 0)
            pltpu.async_copy(abuf, out_hbm.at[bb], osem)

            @pl.when(g < nb - 1)
            def _():
                pltpu.make_async_copy(idx_hbm.at[bb + 1], idx_v, isem).wait()
                pltpu.async_copy(table_hbm.at[idx_v.at[0]], gbuf, gsem)

            return 0

        lax.fori_loop(0, nb, body, 0)
        out_wait(base + nb - 1)

    return k(idx4, table_p)


PLANES_PER_STEP = 16


def _tc_gather(idx2, table, b, t):
    nv, nvv = table.shape

    def body(idx_sref, table_ref, out_ref):
        i = pl.program_id(0)

        def plane(p, _):
            def row(r, _):
                s = idx_sref[(i * PLANES_PER_STEP + p) * t + r]
                out_ref[p, pl.ds(r, 1), :] = table_ref[pl.ds(s, 1), :]
                return 0

            lax.fori_loop(0, t, row, 0)
            return 0

        lax.fori_loop(0, PLANES_PER_STEP, plane, 0)

    grid_spec = pltpu.PrefetchScalarGridSpec(
        num_scalar_prefetch=1,
        grid=(b // PLANES_PER_STEP,),
        in_specs=[pl.BlockSpec((nv, nvv), lambda i, idx_s: (0, 0))],
        out_specs=pl.BlockSpec(
            (PLANES_PER_STEP, t, nvv), lambda i, idx_s: (i, 0, 0)),
    )
    return pl.pallas_call(
        body,
        grid_spec=grid_spec,
        out_shape=jax.ShapeDtypeStruct((b, t, nvv), jnp.float32),
    )(idx2.reshape(-1), table)


def kernel(idx, table):
    b, t = idx.shape
    bs = b // 2
    tp = (t + 7) // 8 * 8
    idx4 = jnp.pad(idx[:bs].reshape(bs, 1, t), ((0, 0), (0, 0), (0, tp - t)))
    table_p = jnp.pad(table, ((0, 0), (0, VPAD - VOCAB)))
    out_a = _sc_gather(idx4, table_p, bs, t, tp)
    out_b = _tc_gather(idx[bs:], table, b - bs, t)
    return jnp.concatenate([out_a, out_b], axis=0)
